# baseline (speedup 1.0000x reference)
import functools

import jax
import jax.numpy as jnp
from jax import lax
from jax.experimental import pallas as pl
from jax.experimental.pallas import tpu as pltpu
from jax.experimental.pallas import tpu_sc as plsc

NC = 2
NS = 16
NW = NC * NS
CH = 88
NBUF = 4


def _sc_count_body(n_pad, k_chunks, rpt,
                   edges_hbm, ones_hbm, zeros_hbm, cnt_hbm,
                   idx_v, ones_v, shared_cnt):
    c = lax.axis_index("c")
    s = lax.axis_index("s")
    w = c * NS + s
    pltpu.sync_copy(edges_hbm.at[w], idx_v)
    pltpu.sync_copy(ones_hbm, ones_v)
    sl = pl.ds(s * rpt, rpt)
    pltpu.sync_copy(zeros_hbm.at[sl], shared_cnt.at[sl])
    plsc.subcore_barrier()

    def chunk(k, carry):
        pltpu.sync_copy(ones_v, shared_cnt.at[idx_v.at[k, 1]], add=True)
        return carry

    lax.fori_loop(0, k_chunks, chunk, 0)
    plsc.subcore_barrier()
    pltpu.sync_copy(shared_cnt.at[sl], cnt_hbm.at[c].at[sl])


def _sc_agg_body(n_pad, k_chunks, rpt,
                 edges_hbm, y_hbm, zeros_hbm, agg_hbm,
                 ib, rows_v, shared_acc, isem, gsem, ssem):
    c = lax.axis_index("c")
    s = lax.axis_index("s")
    w = c * NS + s
    sl = pl.ds(s * rpt, rpt)
    pltpu.sync_copy(zeros_hbm.at[sl], shared_acc.at[sl])
    plsc.subcore_barrier()

    for j in range(min(NBUF - 1, k_chunks)):
        pltpu.async_copy(edges_hbm.at[w, j], ib.at[j], isem)
    for j in range(min(NBUF - 2, k_chunks)):
        pltpu.make_async_copy(edges_hbm.at[w, j], ib.at[j], isem).wait()
        pltpu.async_copy(y_hbm.at[ib.at[j, 0]], rows_v.at[j], gsem)

    def chunk(k, carry):
        kb = lax.rem(k, NBUF)
        pb = lax.rem(k - 1 + NBUF, NBUF)

        pltpu.make_async_copy(
            y_hbm.at[ib.at[kb, 0]], rows_v.at[kb], gsem).wait()

        @pl.when(k >= 1)
        def _():
            pltpu.make_async_copy(
                rows_v.at[pb], shared_acc.at[ib.at[pb, 1]], ssem).wait()

        @pl.when(k + NBUF - 1 < k_chunks)
        def _():
            pltpu.async_copy(edges_hbm.at[w, k + NBUF - 1], ib.at[pb], isem)

        pltpu.async_copy(
            rows_v.at[kb], shared_acc.at[ib.at[kb, 1]], ssem, add=True)

        @pl.when(k + NBUF - 2 < k_chunks)
        def _():
            fb = lax.rem(k + NBUF - 2, NBUF)
            pltpu.make_async_copy(
                edges_hbm.at[w, k + NBUF - 2], ib.at[fb], isem).wait()
            pltpu.async_copy(y_hbm.at[ib.at[fb, 0]], rows_v.at[fb], gsem)

        return carry

    lax.fori_loop(0, k_chunks, chunk, 0)
    lb = (k_chunks - 1) % NBUF
    pltpu.make_async_copy(
        rows_v.at[lb], shared_acc.at[ib.at[lb, 1]], ssem).wait()
    plsc.subcore_barrier()
    pltpu.sync_copy(shared_acc.at[sl], agg_hbm.at[c].at[sl])


def _tc_scale_body(n, x_ref, w_ref, c_ref, y_ref):
    deg = c_ref[0] + c_ref[1] + 1.0
    dis = lax.rsqrt(deg)
    xw = jnp.dot(x_ref[...], w_ref[...], preferred_element_type=jnp.float32)
    y_ref[...] = xw * dis[:n]


def _tc_final_body(n, agg_ref, y_ref, c_ref, bg_ref, wl_ref, bl_ref,
                   h_ref, o_ref):
    deg = c_ref[0] + c_ref[1] + 1.0
    dis = lax.rsqrt(deg)[:n]
    a = agg_ref[0, :n, :] + agg_ref[1, :n, :] + y_ref[...]
    h = jnp.maximum(a * dis + bg_ref[...], 0.0)
    h_ref[...] = h
    o_ref[...] = (
        jnp.dot(h, wl_ref[...], preferred_element_type=jnp.float32)
        + bl_ref[...]
    )


def kernel(x, edge_index, W_gcn, b_gcn, W_lin, b_lin):
    n, d = x.shape
    e = edge_index.shape[1]
    d_hid = W_gcn.shape[1]
    n_cls = W_lin.shape[1]

    n_pad = ((n + 1 + NS * 128 - 1) // (NS * 128)) * (NS * 128)
    rpt = n_pad // NS
    k_chunks = -(-e // (NW * CH))
    e_pad = NW * k_chunks * CH

    src = edge_index[0].astype(jnp.int32)
    dst = edge_index[1].astype(jnp.int32)
    pad = e_pad - e
    src_t = jnp.concatenate([src, jnp.zeros((pad,), jnp.int32)])
    dst_t = jnp.concatenate(
        [dst, n + jnp.arange(pad, dtype=jnp.int32) % (n_pad - n)])
    edges_t = jnp.stack(
        [src_t.reshape(NW, k_chunks, CH), dst_t.reshape(NW, k_chunks, CH)],
        axis=2)

    ones_row = jnp.ones((CH,), jnp.float32)
    zeros_1d = jnp.zeros((n_pad,), jnp.float32)
    zeros_2d = jnp.zeros((n_pad, d_hid), jnp.float32)

    mesh = plsc.VectorSubcoreMesh(
        core_axis_name="c", subcore_axis_name="s",
        num_cores=NC, num_subcores=NS,
    )

    counts = pl.kernel(
        functools.partial(_sc_count_body, n_pad, k_chunks, rpt),
        out_type=jax.ShapeDtypeStruct((NC, n_pad), jnp.float32),
        mesh=mesh,
        scratch_types=[
            pltpu.VMEM((k_chunks, 2, CH), jnp.int32),
            pltpu.VMEM((CH,), jnp.float32),
            pltpu.VMEM_SHARED((n_pad,), jnp.float32),
        ],
    )(edges_t, ones_row, zeros_1d)

    c2 = counts.reshape(NC, n_pad, 1)

    y = pl.pallas_call(
        functools.partial(_tc_scale_body, n),
        out_shape=jax.ShapeDtypeStruct((n, d_hid), jnp.float32),
    )(x, W_gcn, c2)

    agg = pl.kernel(
        functools.partial(_sc_agg_body, n_pad, k_chunks, rpt),
        out_type=jax.ShapeDtypeStruct((NC, n_pad, d_hid), jnp.float32),
        mesh=mesh,
        scratch_types=[
            pltpu.VMEM((NBUF, 2, CH), jnp.int32),
            pltpu.VMEM((NBUF, CH, d_hid), jnp.float32),
            pltpu.VMEM_SHARED((n_pad, d_hid), jnp.float32),
            pltpu.SemaphoreType.DMA,
            pltpu.SemaphoreType.DMA,
            pltpu.SemaphoreType.DMA,
        ],
    )(edges_t, y, zeros_2d)

    n_cls_pad = ((n_cls + 127) // 128) * 128
    wl_pad = jnp.zeros((d_hid, n_cls_pad), jnp.float32).at[:, :n_cls].set(W_lin)
    bl_pad = jnp.zeros((1, n_cls_pad), jnp.float32).at[0, :n_cls].set(b_lin)

    h, out_pad = pl.pallas_call(
        functools.partial(_tc_final_body, n),
        out_shape=(
            jax.ShapeDtypeStruct((n, d_hid), jnp.float32),
            jax.ShapeDtypeStruct((n, n_cls_pad), jnp.float32),
        ),
    )(agg, y, c2, b_gcn.reshape(1, d_hid), wl_pad, bl_pad)

    return (h, out_pad[:, :n_cls])


# speedup vs baseline: 1.1861x; 1.1861x over previous
import functools

import jax
import jax.numpy as jnp
from jax import lax
from jax.experimental import pallas as pl
from jax.experimental.pallas import tpu as pltpu
from jax.experimental.pallas import tpu_sc as plsc

NC = 2
NS = 16
NW = NC * NS
CH = 80
NBUF = 4


def _sc_count_body(n_pad, k_chunks, rpt,
                   edges_hbm, ones_hbm, zeros_hbm, cnt_hbm,
                   idx_v, ones_v, shared_cnt):
    c = lax.axis_index("c")
    s = lax.axis_index("s")
    w = c * NS + s
    pltpu.sync_copy(edges_hbm.at[w], idx_v)
    pltpu.sync_copy(ones_hbm, ones_v)
    sl = pl.ds(s * rpt, rpt)
    pltpu.sync_copy(zeros_hbm.at[sl], shared_cnt.at[sl])
    plsc.subcore_barrier()

    def chunk(k, carry):
        pltpu.sync_copy(ones_v, shared_cnt.at[idx_v.at[k, 1]], add=True)
        return carry

    lax.fori_loop(0, k_chunks, chunk, 0)
    plsc.subcore_barrier()
    pltpu.sync_copy(shared_cnt.at[sl], cnt_hbm.at[c].at[sl])


def _sc_agg_body(n_pad, k_chunks, rpt,
                 edges_hbm, y_hbm, zeros_hbm, agg_hbm,
                 ib, rows_v, shared_acc, isem, gsem, ssem):
    c = lax.axis_index("c")
    s = lax.axis_index("s")
    w = c * NS + s
    sl = pl.ds(s * rpt, rpt)
    pltpu.sync_copy(zeros_hbm.at[sl], shared_acc.at[sl])
    plsc.subcore_barrier()

    for j in range(min(NBUF - 1, k_chunks)):
        pltpu.async_copy(edges_hbm.at[w, j], ib.at[j], isem)
    for j in range(min(NBUF - 2, k_chunks)):
        pltpu.make_async_copy(edges_hbm.at[w, j], ib.at[j], isem).wait()
        pltpu.async_copy(y_hbm.at[ib.at[j, 0]], rows_v.at[j], gsem)

    def chunk(k, carry):
        kb = lax.rem(k, NBUF)
        pb = lax.rem(k - 1 + NBUF, NBUF)

        pltpu.make_async_copy(
            y_hbm.at[ib.at[kb, 0]], rows_v.at[kb], gsem).wait()

        @pl.when(k >= 1)
        def _():
            pltpu.make_async_copy(
                rows_v.at[pb], shared_acc.at[ib.at[pb, 1]], ssem).wait()

        @pl.when(k + NBUF - 1 < k_chunks)
        def _():
            pltpu.async_copy(edges_hbm.at[w, k + NBUF - 1], ib.at[pb], isem)

        pltpu.async_copy(
            rows_v.at[kb], shared_acc.at[ib.at[kb, 1]], ssem, add=True)

        @pl.when(k + NBUF - 2 < k_chunks)
        def _():
            fb = lax.rem(k + NBUF - 2, NBUF)
            pltpu.make_async_copy(
                edges_hbm.at[w, k + NBUF - 2], ib.at[fb], isem).wait()
            pltpu.async_copy(y_hbm.at[ib.at[fb, 0]], rows_v.at[fb], gsem)

        return carry

    lax.fori_loop(0, k_chunks, chunk, 0)
    lb = (k_chunks - 1) % NBUF
    pltpu.make_async_copy(
        rows_v.at[lb], shared_acc.at[ib.at[lb, 1]], ssem).wait()
    plsc.subcore_barrier()
    pltpu.sync_copy(shared_acc.at[sl], agg_hbm.at[c].at[sl])


def _tc_scale_body(n, x_ref, w_ref, c_ref, y_ref):
    deg = c_ref[0] + c_ref[1] + 1.0
    dis = lax.rsqrt(deg)
    xw = jnp.dot(x_ref[...], w_ref[...], preferred_element_type=jnp.float32)
    y_ref[...] = xw * dis[:n]


def _tc_final_body(n, agg_ref, y_ref, c_ref, bg_ref, wl_ref, bl_ref,
                   h_ref, o_ref):
    deg = c_ref[0] + c_ref[1] + 1.0
    dis = lax.rsqrt(deg)[:n]
    a = agg_ref[0, :n, :] + agg_ref[1, :n, :] + y_ref[...]
    h = jnp.maximum(a * dis + bg_ref[...], 0.0)
    h_ref[...] = h
    o_ref[...] = (
        jnp.dot(h, wl_ref[...], preferred_element_type=jnp.float32)
        + bl_ref[...]
    )


def kernel(x, edge_index, W_gcn, b_gcn, W_lin, b_lin):
    n, d = x.shape
    e = edge_index.shape[1]
    d_hid = W_gcn.shape[1]
    n_cls = W_lin.shape[1]

    n_pad = ((n + 1 + NS * 128 - 1) // (NS * 128)) * (NS * 128)
    rpt = n_pad // NS
    k_chunks = -(-e // (NW * CH))
    e_pad = NW * k_chunks * CH

    src = edge_index[0].astype(jnp.int32)
    dst = edge_index[1].astype(jnp.int32)
    pad = e_pad - e
    src_t = jnp.concatenate([src, jnp.zeros((pad,), jnp.int32)])
    dst_t = jnp.concatenate(
        [dst, n + jnp.arange(pad, dtype=jnp.int32) % (n_pad - n)])
    edges_t = jnp.stack(
        [src_t.reshape(NW, k_chunks, CH), dst_t.reshape(NW, k_chunks, CH)],
        axis=2)

    ones_row = jnp.ones((CH,), jnp.float32)
    zeros_1d = jnp.zeros((n_pad,), jnp.float32)
    zeros_2d = jnp.zeros((n_pad, d_hid), jnp.float32)

    mesh = plsc.VectorSubcoreMesh(
        core_axis_name="c", subcore_axis_name="s",
        num_cores=NC, num_subcores=NS,
    )

    counts = pl.kernel(
        functools.partial(_sc_count_body, n_pad, k_chunks, rpt),
        out_type=jax.ShapeDtypeStruct((NC, n_pad), jnp.float32),
        mesh=mesh,
        scratch_types=[
            pltpu.VMEM((k_chunks, 2, CH), jnp.int32),
            pltpu.VMEM((CH,), jnp.float32),
            pltpu.VMEM_SHARED((n_pad,), jnp.float32),
        ],
    )(edges_t, ones_row, zeros_1d)

    c2 = counts.reshape(NC, n_pad, 1)

    y = pl.pallas_call(
        functools.partial(_tc_scale_body, n),
        out_shape=jax.ShapeDtypeStruct((n, d_hid), jnp.float32),
    )(x, W_gcn, c2)

    agg = pl.kernel(
        functools.partial(_sc_agg_body, n_pad, k_chunks, rpt),
        out_type=jax.ShapeDtypeStruct((NC, n_pad, d_hid), jnp.float32),
        mesh=mesh,
        scratch_types=[
            pltpu.VMEM((NBUF, 2, CH), jnp.int32),
            pltpu.VMEM((NBUF, CH, d_hid), jnp.float32),
            pltpu.VMEM_SHARED((n_pad, d_hid), jnp.float32),
            pltpu.SemaphoreType.DMA,
            pltpu.SemaphoreType.DMA,
            pltpu.SemaphoreType.DMA,
        ],
    )(edges_t, y, zeros_2d)

    n_cls_pad = ((n_cls + 127) // 128) * 128
    wl_pad = jnp.zeros((d_hid, n_cls_pad), jnp.float32).at[:, :n_cls].set(W_lin)
    bl_pad = jnp.zeros((1, n_cls_pad), jnp.float32).at[0, :n_cls].set(b_lin)

    h, out_pad = pl.pallas_call(
        functools.partial(_tc_final_body, n),
        out_shape=(
            jax.ShapeDtypeStruct((n, d_hid), jnp.float32),
            jax.ShapeDtypeStruct((n, n_cls_pad), jnp.float32),
        ),
    )(agg, y, c2, b_gcn.reshape(1, d_hid), wl_pad, bl_pad)

    return (h, out_pad[:, :n_cls])
